# Initial kernel scaffold; baseline (speedup 1.0000x reference)
import jax
import jax.numpy as jnp
from jax.experimental import pallas as pl


def kernel(node_feats, node_lens, table):
    raise NotImplementedError("write your pallas kernel here")


# speedup vs baseline: 2.7781x; 2.7781x over previous
import functools

import jax
import jax.numpy as jnp
from jax import lax
from jax.experimental import pallas as pl
from jax.experimental.pallas import tpu as pltpu
from jax.experimental.pallas import tpu_sc as plsc

NC = 2
NS = 16
NW = NC * NS
LANES = 16

CBE = 2
RING = 4


@jax.jit
def kernel(node_feats, node_lens, table):
    del node_lens
    B, H = node_feats.shape
    V, D = table.shape
    assert B % NW == 0 and D % LANES == 0
    BPW = B // NW
    assert BPW % CBE == 0
    NCHUNK = BPW // CBE
    assert NCHUNK % RING == 0
    ROWS = CBE * H
    assert ROWS <= 128
    ND = D // LANES

    feats3 = node_feats.reshape(NW, NCHUNK, ROWS)

    mesh = plsc.VectorSubcoreMesh(core_axis_name="c", subcore_axis_name="s")

    def body(feats_hbm, table_hbm, out_hbm, idx_v, buf_v, outb_v, *sems):
        cid = lax.axis_index("c")
        sid = lax.axis_index("s")
        wid = sid * NC + cid

        pltpu.sync_copy(feats_hbm.at[wid], idx_v)

        def gather(chunk, slot):
            pltpu.async_copy(
                table_hbm.at[idx_v.at[chunk]], buf_v.at[slot], sems[slot]
            )

        def drain(chunk, slot):
            pltpu.make_async_copy(
                table_hbm.at[idx_v.at[chunk]], buf_v.at[slot], sems[slot]
            ).wait()

        for b in range(RING):
            gather(b, b)

        inv = jnp.float32(1.0 / H)

        @pl.loop(0, NCHUNK, step=RING)
        def _(c0):
            for b in range(RING):
                c = c0 + b
                drain(jnp.minimum(c, NCHUNK - 1), b)
                bb = buf_v.at[b]
                for e in range(CBE):
                    row0 = e * H
                    accs = tuple(
                        bb[row0, pl.ds(LANES * k, LANES)] for k in range(ND)
                    )

                    def red(l, a, _row0=row0, _bb=bb):
                        return tuple(
                            a[k] + _bb[_row0 + l, pl.ds(LANES * k, LANES)]
                            for k in range(ND)
                        )

                    accs = lax.fori_loop(1, H, red, accs)
                    orow = c * CBE + e
                    for k in range(ND):
                        outb_v[orow, pl.ds(LANES * k, LANES)] = accs[k] * inv
                cn = c + RING

                @pl.when(cn < NCHUNK)
                def _():
                    gather(jnp.minimum(cn, NCHUNK - 1), b)

        pltpu.sync_copy(outb_v, out_hbm.at[pl.ds(wid * BPW, BPW)])

    run = pl.kernel(
        body,
        out_type=jax.ShapeDtypeStruct((B, D), jnp.float32),
        mesh=mesh,
        compiler_params=pltpu.CompilerParams(use_tc_tiling_on_sc=False),
        scratch_types=[
            pltpu.VMEM((NCHUNK, ROWS), jnp.int32),
            pltpu.VMEM((RING, ROWS, D), jnp.float32),
            pltpu.VMEM((BPW, D), jnp.float32),
        ]
        + [pltpu.SemaphoreType.DMA] * RING,
    )
    return run(feats3, table)


# speedup vs baseline: 2.8094x; 1.0112x over previous
import functools

import jax
import jax.numpy as jnp
from jax import lax
from jax.experimental import pallas as pl
from jax.experimental.pallas import tpu as pltpu
from jax.experimental.pallas import tpu_sc as plsc

NC = 2
NS = 16
NW = NC * NS
LANES = 16

CBE = 2
RING = 4


@jax.jit
def kernel(node_feats, node_lens, table):
    del node_lens
    B, H = node_feats.shape
    V, D = table.shape
    assert B % NW == 0 and D % LANES == 0
    BPW = B // NW
    assert BPW % CBE == 0
    NCHUNK = BPW // CBE
    assert NCHUNK % RING == 0
    ROWS = CBE * H
    assert ROWS <= 128
    ND = D // LANES

    feats3 = node_feats.reshape(NW, NCHUNK, ROWS)

    mesh = plsc.VectorSubcoreMesh(core_axis_name="c", subcore_axis_name="s")

    def body(feats_hbm, table_hbm, out_hbm, idx_v, buf_v, outb_v, *sems):
        cid = lax.axis_index("c")
        sid = lax.axis_index("s")
        wid = sid * NC + cid

        pltpu.sync_copy(feats_hbm.at[wid], idx_v)

        def gather(chunk, slot):
            pltpu.async_copy(
                table_hbm.at[idx_v.at[chunk]], buf_v.at[slot], sems[slot]
            )

        def drain(chunk, slot):
            pltpu.make_async_copy(
                table_hbm.at[idx_v.at[chunk]], buf_v.at[slot], sems[slot]
            ).wait()

        for b in range(RING):
            gather(b, b)

        inv = jnp.float32(1.0 / H)

        @pl.loop(0, NCHUNK, step=RING)
        def _(c0):
            for b in range(RING):
                c = c0 + b
                drain(jnp.minimum(c, NCHUNK - 1), b)
                bb = buf_v.at[b]
                for e in range(CBE):
                    row0 = e * H
                    accs = tuple(
                        bb[row0, pl.ds(LANES * k, LANES)] for k in range(ND)
                    )

                    def red(l, a, _row0=row0, _bb=bb):
                        return tuple(
                            a[k] + _bb[_row0 + l, pl.ds(LANES * k, LANES)]
                            for k in range(ND)
                        )

                    accs = lax.fori_loop(1, H, red, accs, unroll=7)
                    orow = c * CBE + e
                    for k in range(ND):
                        outb_v[orow, pl.ds(LANES * k, LANES)] = accs[k] * inv
                cn = c + RING

                @pl.when(cn < NCHUNK)
                def _():
                    gather(jnp.minimum(cn, NCHUNK - 1), b)

        pltpu.sync_copy(outb_v, out_hbm.at[pl.ds(wid * BPW, BPW)])

    run = pl.kernel(
        body,
        out_type=jax.ShapeDtypeStruct((B, D), jnp.float32),
        mesh=mesh,
        compiler_params=pltpu.CompilerParams(use_tc_tiling_on_sc=False),
        scratch_types=[
            pltpu.VMEM((NCHUNK, ROWS), jnp.int32),
            pltpu.VMEM((RING, ROWS, D), jnp.float32),
            pltpu.VMEM((BPW, D), jnp.float32),
        ]
        + [pltpu.SemaphoreType.DMA] * RING,
    )
    return run(feats3, table)
